# baseline (speedup 1.0000x reference)
import functools

import jax
import jax.numpy as jnp
from jax import lax
from jax.experimental import pallas as pl

B, C, H, W = 4, 96, 384, 384
HW = H * W
S = HW // 128
TOP_K, TOP_L, TOP_I = 512, 128, 128
N = TOP_K + TOP_L + TOP_I
MAXBITS = 0x3F800000


def _sel_kernel(p_ref, a_ref, thr_ref):
  p = p_ref[...]
  conf = jnp.abs(2.0 * p - 1.0)
  a_ref[...] = conf
  unc_bits = lax.bitcast_convert_type(1.0 - conf, jnp.int32)
  p_bits = lax.bitcast_convert_type(p, jnp.int32)
  idx = (
      jax.lax.broadcasted_iota(jnp.int32, (S, 128), 0) * 128
      + jax.lax.broadcasted_iota(jnp.int32, (S, 128), 1)
  )[None]

  def cnt(pred):
    return jnp.sum(pred.astype(jnp.int32), axis=(1, 2))

  def topk_search(bits, k, largest):
    lo = jnp.zeros((B,), jnp.int32)
    hi = jnp.full((B,), MAXBITS, jnp.int32)

    def body(_, lohi):
      lo, hi = lohi
      if largest:
        mid = lo + (hi - lo + 1) // 2
        ok = cnt(bits >= mid[:, None, None]) >= k
        return jnp.where(ok, mid, lo), jnp.where(ok, hi, mid - 1)
      else:
        mid = lo + (hi - lo) // 2
        ok = cnt(bits <= mid[:, None, None]) >= k
        return jnp.where(ok, lo, mid + 1), jnp.where(ok, mid, hi)

    lo, hi = lax.fori_loop(0, 30, body, (lo, hi))
    t = lo if largest else hi
    if largest:
      g = cnt(bits > t[:, None, None])
    else:
      g = cnt(bits < t[:, None, None])
    eq = bits == t[:, None, None]
    lo2 = jnp.zeros((B,), jnp.int32)
    hi2 = jnp.full((B,), HW - 1, jnp.int32)

    def body2(_, lohi):
      lo2, hi2 = lohi
      mid = lo2 + (hi2 - lo2) // 2
      c = g + cnt(eq & (idx <= mid[:, None, None]))
      ok = c >= k
      return jnp.where(ok, lo2, mid + 1), jnp.where(ok, mid, hi2)

    lo2, hi2 = lax.fori_loop(0, 18, body2, (lo2, hi2))
    return t, hi2

  tk, ik = topk_search(unc_bits, TOP_K, True)
  tl, il = topk_search(p_bits, TOP_L, True)
  ti, ii = topk_search(p_bits, TOP_I, False)

  lane = jax.lax.broadcasted_iota(jnp.int32, (B, 128), 1)
  out = jnp.zeros((B, 128), jnp.int32)
  for j, v in enumerate([tk, ik, tl, il, ti, ii]):
    out = jnp.where(lane == j, v[:, None], out)
  thr_ref[...] = out


def _mlp_kernel(ft_ref, w1_ref, b1_ref, w2_ref, b2_ref, psel_ref, vals_ref):
  ft = ft_ref[0]
  h = lax.dot_general(ft, w1_ref[...], (((0,), (0,)), ((), ())),
                      preferred_element_type=jnp.float32)
  h = jnp.maximum(h + b1_ref[...], 0.0)
  lg = lax.dot_general(w2_ref[...], h, (((1,), (1,)), ((), ())),
                       preferred_element_type=jnp.float32)
  prob = jax.nn.sigmoid(lg + b2_ref[...])
  psel = psel_ref[0]
  vals_ref[0] = jnp.abs(2.0 * psel - 1.0) * psel + prob


def _select_list(bits, t, i, largest, k):
  idx = jnp.arange(HW, dtype=jnp.int32)[None]
  if largest:
    sel = (bits > t[:, None]) | ((bits == t[:, None]) & (idx <= i[:, None]))
  else:
    sel = (bits < t[:, None]) | ((bits == t[:, None]) & (idx <= i[:, None]))
  pos = jnp.cumsum(sel.astype(jnp.int32), axis=1) - 1
  pos = jnp.where(sel, pos, k)
  out = jnp.zeros((B, k), jnp.int32)
  bb = jnp.arange(B, dtype=jnp.int32)[:, None]
  return out.at[bb, pos].set(jnp.broadcast_to(idx, (B, HW)), mode="drop")


@jax.jit
def kernel(p_coarse_mask, feature_map, W1, b1, W2, b2):
  p2 = p_coarse_mask.reshape(B, S, 128)
  a3, thr = pl.pallas_call(
      _sel_kernel,
      out_shape=(
          jax.ShapeDtypeStruct((B, S, 128), jnp.float32),
          jax.ShapeDtypeStruct((B, 128), jnp.int32),
      ),
  )(p2)
  A = a3.reshape(B, 1, H, W)

  pf = p_coarse_mask.reshape(B, HW)
  unc_bits = lax.bitcast_convert_type(1.0 - jnp.abs(2.0 * pf - 1.0), jnp.int32)
  p_bits = lax.bitcast_convert_type(pf, jnp.int32)
  pix_k = _select_list(unc_bits, thr[:, 0], thr[:, 1], True, TOP_K)
  pix_l = _select_list(p_bits, thr[:, 2], thr[:, 3], True, TOP_L)
  pix_i = _select_list(p_bits, thr[:, 4], thr[:, 5], False, TOP_I)
  pix = jnp.concatenate([pix_k, pix_l, pix_i], axis=1)

  f2 = feature_map.reshape(B, C, HW)
  featsT = jnp.take_along_axis(f2, pix[:, None, :], axis=2)
  psel = jnp.take_along_axis(pf, pix, axis=1).reshape(B, 1, N)

  vals = pl.pallas_call(
      _mlp_kernel,
      grid=(B,),
      in_specs=[
          pl.BlockSpec((1, C, N), lambda b: (b, 0, 0)),
          pl.BlockSpec((C, 256), lambda b: (0, 0)),
          pl.BlockSpec((1, 256), lambda b: (0, 0)),
          pl.BlockSpec((1, 256), lambda b: (0, 0)),
          pl.BlockSpec((1, 1), lambda b: (0, 0)),
          pl.BlockSpec((1, 1, N), lambda b: (b, 0, 0)),
      ],
      out_specs=pl.BlockSpec((1, 1, N), lambda b: (b, 0, 0)),
      out_shape=jax.ShapeDtypeStruct((B, 1, N), jnp.float32),
  )(featsT, W1, b1.reshape(1, 256), W2.reshape(1, 256), b2.reshape(1, 1),
    psel)
  vals = vals.reshape(B, N)

  base = (A.reshape(B, HW) * pf)
  bb = jnp.arange(B, dtype=jnp.int32)[:, None]
  yu = base.at[bb, pix].set(vals).reshape(B, 1, H, W)
  return (A, yu)


# speedup vs baseline: 15.9728x; 15.9728x over previous
import functools

import jax
import jax.numpy as jnp
from jax import lax
from jax.experimental import pallas as pl
from jax.experimental.pallas import tpu as pltpu
from jax.experimental.pallas import tpu_sc as plsc

B, C, H, W = 4, 96, 384, 384
HW = H * W
S = HW // 128
TOP_K, TOP_L, TOP_I = 512, 128, 128
N = TOP_K + TOP_L + TOP_I
NPAD = N + 16
MAXBITS = 0x3F800000
NTILES = 16
CHUNK = HW // NTILES
NV = CHUNK // 16
SLOT_BASE = (0, TOP_K, TOP_K + TOP_L)
SEL_COUNT = (TOP_K, TOP_L, TOP_I)


def _sel_kernel(p_ref, a_ref, thr_ref, pref_ref):
  p = p_ref[...]
  conf = jnp.abs(2.0 * p - 1.0)
  a_ref[...] = conf
  unc_bits = lax.bitcast_convert_type(1.0 - conf, jnp.int32)
  p_bits = lax.bitcast_convert_type(p, jnp.int32)
  idx = (
      jax.lax.broadcasted_iota(jnp.int32, (S, 128), 0) * 128
      + jax.lax.broadcasted_iota(jnp.int32, (S, 128), 1)
  )[None]

  def cnt(pred):
    return jnp.sum(pred.astype(jnp.int32), axis=(1, 2))

  def topk_search(bits, k, largest):
    lo = jnp.zeros((B,), jnp.int32)
    hi = jnp.full((B,), MAXBITS, jnp.int32)

    def body(_, lohi):
      lo, hi = lohi
      if largest:
        mid = lo + (hi - lo + 1) // 2
        ok = cnt(bits >= mid[:, None, None]) >= k
        return jnp.where(ok, mid, lo), jnp.where(ok, hi, mid - 1)
      else:
        mid = lo + (hi - lo) // 2
        ok = cnt(bits <= mid[:, None, None]) >= k
        return jnp.where(ok, lo, mid + 1), jnp.where(ok, mid, hi)

    lo, hi = lax.fori_loop(0, 30, body, (lo, hi))
    t = lo if largest else hi
    if largest:
      g = cnt(bits > t[:, None, None])
    else:
      g = cnt(bits < t[:, None, None])
    eq = bits == t[:, None, None]
    lo2 = jnp.zeros((B,), jnp.int32)
    hi2 = jnp.full((B,), HW - 1, jnp.int32)

    def body2(_, lohi):
      lo2, hi2 = lohi
      mid = lo2 + (hi2 - lo2) // 2
      ok = g + cnt(eq & (idx <= mid[:, None, None])) >= k
      return jnp.where(ok, lo2, mid + 1), jnp.where(ok, mid, hi2)

    lo2, hi2 = lax.fori_loop(0, 18, body2, (lo2, hi2))
    return t, hi2

  tk, ik = topk_search(unc_bits, TOP_K, True)
  tl, il = topk_search(p_bits, TOP_L, True)
  ti, ii = topk_search(p_bits, TOP_I, False)

  lane = jax.lax.broadcasted_iota(jnp.int32, (B, 128), 1)
  out = jnp.zeros((B, 128), jnp.int32)
  for j, v in enumerate([tk, ik, tl, il, ti, ii]):
    out = jnp.where(lane == j, v[:, None], out)
  thr_ref[...] = out

  def chunk_prefix(bits, t, i, largest):
    if largest:
      sel = (bits > t[:, None, None]) | ((bits == t[:, None, None])
                                         & (idx <= i[:, None, None]))
    else:
      sel = (bits < t[:, None, None]) | ((bits == t[:, None, None])
                                         & (idx <= i[:, None, None]))
    seli = sel.astype(jnp.int32)
    rows_per_chunk = CHUNK // 128
    out = jnp.zeros((B, 128), jnp.int32)
    run = jnp.zeros((B,), jnp.int32)
    for t in range(NTILES):
      out = jnp.where(lane == t, run[:, None], out)
      run = run + jnp.sum(
          seli[:, t * rows_per_chunk:(t + 1) * rows_per_chunk, :], axis=(1, 2))
    return out

  pref_ref[...] = jnp.concatenate([
      chunk_prefix(unc_bits, tk, ik, True),
      chunk_prefix(p_bits, tl, il, True),
      chunk_prefix(p_bits, ti, ii, False),
  ], axis=1)


def _compact_kernel(p_hbm, thr_hbm, pref_hbm, pix_hbm, psel_hbm, pvm, thr_s,
                    pref_s, ibuf0, ibuf1, ibuf2, pbuf0, pbuf1, pbuf2):
  core = lax.axis_index("c")
  tile = lax.axis_index("s")
  lanes = lax.iota(jnp.int32, 16)
  ibufs = (ibuf0, ibuf1, ibuf2)
  pbufs = (pbuf0, pbuf1, pbuf2)

  for b_local in range(2):
    b = 2 * core + b_local
    pltpu.sync_copy(p_hbm.at[b, pl.ds(tile * CHUNK, CHUNK)], pvm)
    pltpu.sync_copy(thr_hbm.at[b], thr_s)
    thr_vec = thr_s[pl.ds(0, 16)]
    tk, ik = thr_vec[0], thr_vec[1]
    tl, il = thr_vec[2], thr_vec[3]
    ti, ii = thr_vec[4], thr_vec[5]
    tile_base = tile * CHUNK

    def scan_body(i, offs):
      o0, o1, o2 = offs
      v = pvm[pl.ds(i * 16, 16)]
      pb = plsc.bitcast(v, jnp.int32)
      ub = plsc.bitcast(1.0 - jnp.abs(2.0 * v - 1.0), jnp.int32)
      gi = tile_base + i * 16 + lanes
      m0 = (ub > tk) | ((ub == tk) & (gi <= ik))
      m1 = (pb > tl) | ((pb == tl) & (gi <= il))
      m2 = (pb < ti) | ((pb == ti) & (gi <= ii))
      plsc.store_compressed(ibuf0.at[pl.ds(o0, 16)], gi, mask=m0)
      plsc.store_compressed(pbuf0.at[pl.ds(o0, 16)], v, mask=m0)
      plsc.store_compressed(ibuf1.at[pl.ds(o1, 16)], gi, mask=m1)
      plsc.store_compressed(pbuf1.at[pl.ds(o1, 16)], v, mask=m1)
      plsc.store_compressed(ibuf2.at[pl.ds(o2, 16)], gi, mask=m2)
      plsc.store_compressed(pbuf2.at[pl.ds(o2, 16)], v, mask=m2)
      o0 = o0 + jnp.sum(m0.astype(jnp.int32))
      o1 = o1 + jnp.sum(m1.astype(jnp.int32))
      o2 = o2 + jnp.sum(m2.astype(jnp.int32))
      return (o0, o1, o2)

    z = jnp.int32(0)
    counts = lax.fori_loop(0, NV, scan_body, (z, z, z))

    for s in range(3):
      pltpu.sync_copy(pref_hbm.at[pl.ds((b * 3 + s) * 128, 128)], pref_s)
      pre_vec = pref_s[pl.ds(0, 16)]
      prefix = jnp.sum(jnp.where(lanes == tile, pre_vec, 0))
      start = b * NPAD + SLOT_BASE[s] + prefix
      dump = b * NPAD + N + lanes
      cnt_s = counts[s]

      def wr_body(j, _, s=s, start=start, dump=dump, cnt_s=cnt_s):
        lanepos = j * 16 + lanes
        tgt = jnp.where(lanepos < cnt_s, start + lanepos, dump)
        pltpu.sync_copy(ibufs[s].at[pl.ds(j * 16, 16)], pix_hbm.at[tgt])
        pltpu.sync_copy(pbufs[s].at[pl.ds(j * 16, 16)], psel_hbm.at[tgt])
        return 0

      lax.fori_loop(0, (cnt_s + 15) // 16, wr_body, 0)


def _gather_kernel(pix_hbm, feat_hbm, ft_hbm, idxbuf, fvm):
  core = lax.axis_index("c")
  tile = lax.axis_index("s")
  for b_local in range(2):
    b = 2 * core + b_local
    for j in range(N // 128):
      pltpu.sync_copy(pix_hbm.at[pl.ds(b * NPAD + 128 * j, 128)],
                      idxbuf.at[j])
    for c_local in range(C // NTILES):
      ch = c_local * NTILES + tile
      base_off = (b * C + ch) * HW
      for j in range(N // 128):
        pltpu.sync_copy(
            feat_hbm.at[pl.ds(base_off, HW)].at[idxbuf.at[j]],
            fvm.at[pl.ds(j * 128, 128)])
      pltpu.sync_copy(fvm, ft_hbm.at[b, ch])


def _mlp_kernel(ft_ref, w1_ref, b1_ref, w2_ref, b2_ref, psel_ref, vals_ref):
  ft = ft_ref[0]
  h = lax.dot_general(ft, w1_ref[...], (((0,), (0,)), ((), ())),
                      precision=lax.Precision.HIGHEST,
                      preferred_element_type=jnp.float32)
  h = jnp.maximum(h + b1_ref[...], 0.0)
  lg = lax.dot_general(w2_ref[...], h, (((1,), (1,)), ((), ())),
                       precision=lax.Precision.HIGHEST,
                       preferred_element_type=jnp.float32)
  prob = jax.nn.sigmoid(lg + b2_ref[...])
  psel = psel_ref[0]
  vals_ref[0] = jnp.abs(2.0 * psel - 1.0) * psel + prob


def _scatter_kernel(p_hbm, pix_hbm, vals_hbm, yu_hbm, pvm, pixvm, valsvm,
                    idxbuf):
  core = lax.axis_index("c")
  tile = lax.axis_index("s")
  lanes = lax.iota(jnp.int32, 16)

  for b_local in range(2):
    b = 2 * core + b_local
    pltpu.sync_copy(p_hbm.at[pl.ds(b * HW + tile * CHUNK, CHUNK)], pvm)

    @pl.loop(0, NV)
    def _(i):
      v = pvm[pl.ds(i * 16, 16)]
      pvm[pl.ds(i * 16, 16)] = jnp.abs(2.0 * v - 1.0) * v

    pltpu.sync_copy(pvm, yu_hbm.at[pl.ds(b * HW + tile * CHUNK, CHUNK)])

  plsc.subcore_barrier()

  NS = N // NTILES
  for b_local in range(2):
    b = 2 * core + b_local
    pltpu.sync_copy(pix_hbm.at[pl.ds(b * NPAD + tile * NS, NS)], pixvm)
    pltpu.sync_copy(vals_hbm.at[pl.ds(b * N + tile * NS, NS)], valsvm)
    for l in range(NS // 16):
      tgt = pixvm[pl.ds(l * 16, 16)] + b * HW
      pltpu.sync_copy(valsvm.at[pl.ds(l * 16, 16)], yu_hbm.at[tgt])


_sc_mesh = plsc.VectorSubcoreMesh(core_axis_name="c", subcore_axis_name="s",
                                  num_cores=2, num_subcores=16)
_sc_params = pltpu.CompilerParams(needs_layout_passes=False)


@jax.jit
def kernel(p_coarse_mask, feature_map, W1, b1, W2, b2):
  p2 = p_coarse_mask.reshape(B, S, 128)
  a3, thr, pref = pl.pallas_call(
      _sel_kernel,
      out_shape=(
          jax.ShapeDtypeStruct((B, S, 128), jnp.float32),
          jax.ShapeDtypeStruct((B, 128), jnp.int32),
          jax.ShapeDtypeStruct((B, 384), jnp.int32),
      ),
  )(p2)
  A = a3.reshape(B, 1, H, W)

  pf = p_coarse_mask.reshape(B, HW)
  feat_flat = feature_map.reshape(B * C * HW)

  compact = pl.kernel(
      _compact_kernel,
      out_type=(
          jax.ShapeDtypeStruct((B * NPAD,), jnp.int32),
          jax.ShapeDtypeStruct((B * NPAD,), jnp.float32),
      ),
      mesh=_sc_mesh,
      scratch_types=[
          pltpu.VMEM((CHUNK,), jnp.float32),
          pltpu.VMEM((128,), jnp.int32),
          pltpu.VMEM((128,), jnp.int32),
          pltpu.VMEM((TOP_K + 16,), jnp.int32),
          pltpu.VMEM((TOP_L + 16,), jnp.int32),
          pltpu.VMEM((TOP_I + 16,), jnp.int32),
          pltpu.VMEM((TOP_K + 16,), jnp.float32),
          pltpu.VMEM((TOP_L + 16,), jnp.float32),
          pltpu.VMEM((TOP_I + 16,), jnp.float32),
      ],
      compiler_params=_sc_params,
  )
  pix_flat, psel_flat = compact(pf, thr, pref.reshape(B * 384))

  gather = pl.kernel(
      _gather_kernel,
      out_type=jax.ShapeDtypeStruct((B, C, N), jnp.float32),
      mesh=_sc_mesh,
      scratch_types=[
          pltpu.VMEM((N // 128, 128), jnp.int32),
          pltpu.VMEM((N,), jnp.float32),
      ],
      compiler_params=_sc_params,
  )
  featsT = gather(pix_flat, feat_flat)

  psel = psel_flat.reshape(B, NPAD)[:, :N].reshape(B, 1, N)
  vals = pl.pallas_call(
      _mlp_kernel,
      grid=(B,),
      in_specs=[
          pl.BlockSpec((1, C, N), lambda b: (b, 0, 0)),
          pl.BlockSpec((C, 256), lambda b: (0, 0)),
          pl.BlockSpec((1, 256), lambda b: (0, 0)),
          pl.BlockSpec((1, 256), lambda b: (0, 0)),
          pl.BlockSpec((1, 1), lambda b: (0, 0)),
          pl.BlockSpec((1, 1, N), lambda b: (b, 0, 0)),
      ],
      out_specs=pl.BlockSpec((1, 1, N), lambda b: (b, 0, 0)),
      out_shape=jax.ShapeDtypeStruct((B, 1, N), jnp.float32),
  )(featsT, W1, b1.reshape(1, 256), W2.reshape(1, 256), b2.reshape(1, 1),
    psel)

  scatter = pl.kernel(
      _scatter_kernel,
      out_type=jax.ShapeDtypeStruct((B * HW,), jnp.float32),
      mesh=_sc_mesh,
      scratch_types=[
          pltpu.VMEM((CHUNK,), jnp.float32),
          pltpu.VMEM((N // NTILES,), jnp.int32),
          pltpu.VMEM((N // NTILES,), jnp.float32),
          pltpu.VMEM((N // NTILES,), jnp.int32),
      ],
      compiler_params=_sc_params,
  )
  yu = scatter(pf.reshape(B * HW), pix_flat, vals.reshape(B * N))
  return (A, yu.reshape(B, 1, H, W))


# speedup vs baseline: 15.9815x; 1.0005x over previous
import functools

import jax
import jax.numpy as jnp
from jax import lax
from jax.experimental import pallas as pl
from jax.experimental.pallas import tpu as pltpu
from jax.experimental.pallas import tpu_sc as plsc

B, C, H, W = 4, 96, 384, 384
HW = H * W
S = HW // 128
TOP_K, TOP_L, TOP_I = 512, 128, 128
N = TOP_K + TOP_L + TOP_I
NPAD = N + 16
MAXBITS = 0x3F800000
NTILES = 16
CHUNK = HW // NTILES
NV = CHUNK // 16
SLOT_BASE = (0, TOP_K, TOP_K + TOP_L)
SEL_COUNT = (TOP_K, TOP_L, TOP_I)


def _sel_kernel(p_ref, a_ref, thr_ref, pref_ref):
  p = p_ref[...]
  conf = jnp.abs(2.0 * p - 1.0)
  a_ref[...] = conf
  unc_bits = lax.bitcast_convert_type(1.0 - conf, jnp.int32)
  p_bits = lax.bitcast_convert_type(p, jnp.int32)
  idx = (
      jax.lax.broadcasted_iota(jnp.int32, (S, 128), 0) * 128
      + jax.lax.broadcasted_iota(jnp.int32, (S, 128), 1)
  )[None]

  def cnt(pred):
    return jnp.sum(pred.astype(jnp.int32), axis=(1, 2))

  def topk_search(bits, k, largest):
    lo = jnp.zeros((B,), jnp.int32)
    hi = jnp.full((B,), MAXBITS, jnp.int32)

    def body(_, lohi):
      lo, hi = lohi
      if largest:
        mid = lo + (hi - lo + 1) // 2
        ok = cnt(bits >= mid[:, None, None]) >= k
        return jnp.where(ok, mid, lo), jnp.where(ok, hi, mid - 1)
      else:
        mid = lo + (hi - lo) // 2
        ok = cnt(bits <= mid[:, None, None]) >= k
        return jnp.where(ok, lo, mid + 1), jnp.where(ok, mid, hi)

    lo, hi = lax.fori_loop(0, 30, body, (lo, hi))
    t = lo if largest else hi
    if largest:
      g = cnt(bits > t[:, None, None])
    else:
      g = cnt(bits < t[:, None, None])
    eq = bits == t[:, None, None]
    lo2 = jnp.zeros((B,), jnp.int32)
    hi2 = jnp.full((B,), HW - 1, jnp.int32)

    def body2(_, lohi):
      lo2, hi2 = lohi
      mid = lo2 + (hi2 - lo2) // 2
      ok = g + cnt(eq & (idx <= mid[:, None, None])) >= k
      return jnp.where(ok, lo2, mid + 1), jnp.where(ok, mid, hi2)

    lo2, hi2 = lax.fori_loop(0, 18, body2, (lo2, hi2))
    return t, hi2

  tk, ik = topk_search(unc_bits, TOP_K, True)
  tl, il = topk_search(p_bits, TOP_L, True)
  ti, ii = topk_search(p_bits, TOP_I, False)

  lane = jax.lax.broadcasted_iota(jnp.int32, (B, 128), 1)
  out = jnp.zeros((B, 128), jnp.int32)
  for j, v in enumerate([tk, ik, tl, il, ti, ii]):
    out = jnp.where(lane == j, v[:, None], out)
  thr_ref[...] = out

  def chunk_prefix(bits, t, i, largest):
    if largest:
      sel = (bits > t[:, None, None]) | ((bits == t[:, None, None])
                                         & (idx <= i[:, None, None]))
    else:
      sel = (bits < t[:, None, None]) | ((bits == t[:, None, None])
                                         & (idx <= i[:, None, None]))
    seli = sel.astype(jnp.int32)
    rows_per_chunk = CHUNK // 128
    out = jnp.zeros((B, 128), jnp.int32)
    run = jnp.zeros((B,), jnp.int32)
    for t in range(NTILES):
      out = jnp.where(lane == t, run[:, None], out)
      run = run + jnp.sum(
          seli[:, t * rows_per_chunk:(t + 1) * rows_per_chunk, :], axis=(1, 2))
    return out

  pref_ref[...] = jnp.concatenate([
      chunk_prefix(unc_bits, tk, ik, True),
      chunk_prefix(p_bits, tl, il, True),
      chunk_prefix(p_bits, ti, ii, False),
  ], axis=1)


def _compact_kernel(p_hbm, thr_hbm, pref_hbm, pix_hbm, psel_hbm, pvm, thr_s,
                    pref_s, ibuf0, ibuf1, ibuf2, pbuf0, pbuf1, pbuf2, offs_s):
  core = lax.axis_index("c")
  tile = lax.axis_index("s")
  lanes = lax.iota(jnp.int32, 16)
  ibufs = (ibuf0, ibuf1, ibuf2)
  pbufs = (pbuf0, pbuf1, pbuf2)

  for b_local in range(2):
    b = 2 * core + b_local
    pltpu.sync_copy(p_hbm.at[b, pl.ds(tile * CHUNK, CHUNK)], pvm)
    pltpu.sync_copy(thr_hbm.at[b], thr_s)
    thr_vec = thr_s[pl.ds(0, 16)]
    tk, ik = thr_vec[0], thr_vec[1]
    tl, il = thr_vec[2], thr_vec[3]
    ti, ii = thr_vec[4], thr_vec[5]
    tile_base = tile * CHUNK

    offs_s[0] = 0
    offs_s[1] = 0
    offs_s[2] = 0

    @pl.loop(0, NV)
    def _(i):
      v = pvm[pl.ds(i * 16, 16)]
      pb = plsc.bitcast(v, jnp.int32)
      ub = plsc.bitcast(1.0 - jnp.abs(2.0 * v - 1.0), jnp.int32)
      gi = tile_base + i * 16 + lanes
      m0 = (ub > tk) | ((ub == tk) & (gi <= ik))
      m1 = (pb > tl) | ((pb == tl) & (gi <= il))
      m2 = (pb < ti) | ((pb == ti) & (gi <= ii))
      nz = plsc.all_reduce_population_count(m0 | m1 | m2)

      @pl.when(nz[0] > 0)
      def _():
        o0, o1, o2 = offs_s[0], offs_s[1], offs_s[2]
        plsc.store_compressed(ibuf0.at[pl.ds(o0, 16)], gi, mask=m0)
        plsc.store_compressed(pbuf0.at[pl.ds(o0, 16)], v, mask=m0)
        plsc.store_compressed(ibuf1.at[pl.ds(o1, 16)], gi, mask=m1)
        plsc.store_compressed(pbuf1.at[pl.ds(o1, 16)], v, mask=m1)
        plsc.store_compressed(ibuf2.at[pl.ds(o2, 16)], gi, mask=m2)
        plsc.store_compressed(pbuf2.at[pl.ds(o2, 16)], v, mask=m2)
        offs_s[0] = o0 + plsc.all_reduce_population_count(m0)[0]
        offs_s[1] = o1 + plsc.all_reduce_population_count(m1)[0]
        offs_s[2] = o2 + plsc.all_reduce_population_count(m2)[0]

    counts = (offs_s[0], offs_s[1], offs_s[2])

    for s in range(3):
      pltpu.sync_copy(pref_hbm.at[pl.ds((b * 3 + s) * 128, 128)], pref_s)
      pre_vec = pref_s[pl.ds(0, 16)]
      prefix = jnp.sum(jnp.where(lanes == tile, pre_vec, 0))
      start = b * NPAD + SLOT_BASE[s] + prefix
      dump = b * NPAD + N + lanes
      cnt_s = counts[s]

      def wr_body(j, _, s=s, start=start, dump=dump, cnt_s=cnt_s):
        lanepos = j * 16 + lanes
        tgt = jnp.where(lanepos < cnt_s, start + lanepos, dump)
        pltpu.sync_copy(ibufs[s].at[pl.ds(j * 16, 16)], pix_hbm.at[tgt])
        pltpu.sync_copy(pbufs[s].at[pl.ds(j * 16, 16)], psel_hbm.at[tgt])
        return 0

      lax.fori_loop(0, (cnt_s + 15) // 16, wr_body, 0)


def _gather_kernel(pix_hbm, feat_hbm, ft_hbm, idxbuf, fvm):
  core = lax.axis_index("c")
  tile = lax.axis_index("s")
  for b_local in range(2):
    b = 2 * core + b_local
    for j in range(N // 128):
      pltpu.sync_copy(pix_hbm.at[pl.ds(b * NPAD + 128 * j, 128)],
                      idxbuf.at[j])
    for c_local in range(C // NTILES):
      ch = c_local * NTILES + tile
      base_off = (b * C + ch) * HW
      for j in range(N // 128):
        pltpu.sync_copy(
            feat_hbm.at[pl.ds(base_off, HW)].at[idxbuf.at[j]],
            fvm.at[pl.ds(j * 128, 128)])
      pltpu.sync_copy(fvm, ft_hbm.at[b, ch])


def _mlp_kernel(ft_ref, w1_ref, b1_ref, w2_ref, b2_ref, psel_ref, vals_ref):
  ft = ft_ref[0]
  h = lax.dot_general(ft, w1_ref[...], (((0,), (0,)), ((), ())),
                      precision=lax.Precision.HIGHEST,
                      preferred_element_type=jnp.float32)
  h = jnp.maximum(h + b1_ref[...], 0.0)
  lg = lax.dot_general(w2_ref[...], h, (((1,), (1,)), ((), ())),
                       precision=lax.Precision.HIGHEST,
                       preferred_element_type=jnp.float32)
  prob = jax.nn.sigmoid(lg + b2_ref[...])
  psel = psel_ref[0]
  vals_ref[0] = jnp.abs(2.0 * psel - 1.0) * psel + prob


def _scatter_kernel(p_hbm, pix_hbm, vals_hbm, yu_hbm, pvm, pixvm, valsvm,
                    idxbuf):
  core = lax.axis_index("c")
  tile = lax.axis_index("s")
  lanes = lax.iota(jnp.int32, 16)

  for b_local in range(2):
    b = 2 * core + b_local
    pltpu.sync_copy(p_hbm.at[pl.ds(b * HW + tile * CHUNK, CHUNK)], pvm)

    @pl.loop(0, NV)
    def _(i):
      v = pvm[pl.ds(i * 16, 16)]
      pvm[pl.ds(i * 16, 16)] = jnp.abs(2.0 * v - 1.0) * v

    pltpu.sync_copy(pvm, yu_hbm.at[pl.ds(b * HW + tile * CHUNK, CHUNK)])

  plsc.subcore_barrier()

  NS = N // NTILES
  for b_local in range(2):
    b = 2 * core + b_local
    pltpu.sync_copy(pix_hbm.at[pl.ds(b * NPAD + tile * NS, NS)], pixvm)
    pltpu.sync_copy(vals_hbm.at[pl.ds(b * N + tile * NS, NS)], valsvm)
    for l in range(NS // 16):
      tgt = pixvm[pl.ds(l * 16, 16)] + b * HW
      pltpu.sync_copy(valsvm.at[pl.ds(l * 16, 16)], yu_hbm.at[tgt])


_sc_mesh = plsc.VectorSubcoreMesh(core_axis_name="c", subcore_axis_name="s",
                                  num_cores=2, num_subcores=16)
_sc_params = pltpu.CompilerParams(needs_layout_passes=False)


@jax.jit
def kernel(p_coarse_mask, feature_map, W1, b1, W2, b2):
  p2 = p_coarse_mask.reshape(B, S, 128)
  a3, thr, pref = pl.pallas_call(
      _sel_kernel,
      out_shape=(
          jax.ShapeDtypeStruct((B, S, 128), jnp.float32),
          jax.ShapeDtypeStruct((B, 128), jnp.int32),
          jax.ShapeDtypeStruct((B, 384), jnp.int32),
      ),
  )(p2)
  A = a3.reshape(B, 1, H, W)

  pf = p_coarse_mask.reshape(B, HW)
  feat_flat = feature_map.reshape(B * C * HW)

  compact = pl.kernel(
      _compact_kernel,
      out_type=(
          jax.ShapeDtypeStruct((B * NPAD,), jnp.int32),
          jax.ShapeDtypeStruct((B * NPAD,), jnp.float32),
      ),
      mesh=_sc_mesh,
      scratch_types=[
          pltpu.VMEM((CHUNK,), jnp.float32),
          pltpu.VMEM((128,), jnp.int32),
          pltpu.VMEM((128,), jnp.int32),
          pltpu.VMEM((TOP_K + 16,), jnp.int32),
          pltpu.VMEM((TOP_L + 16,), jnp.int32),
          pltpu.VMEM((TOP_I + 16,), jnp.int32),
          pltpu.VMEM((TOP_K + 16,), jnp.float32),
          pltpu.VMEM((TOP_L + 16,), jnp.float32),
          pltpu.VMEM((TOP_I + 16,), jnp.float32),
          pltpu.SMEM((8,), jnp.int32),
      ],
      compiler_params=_sc_params,
  )
  pix_flat, psel_flat = compact(pf, thr, pref.reshape(B * 384))

  gather = pl.kernel(
      _gather_kernel,
      out_type=jax.ShapeDtypeStruct((B, C, N), jnp.float32),
      mesh=_sc_mesh,
      scratch_types=[
          pltpu.VMEM((N // 128, 128), jnp.int32),
          pltpu.VMEM((N,), jnp.float32),
      ],
      compiler_params=_sc_params,
  )
  featsT = gather(pix_flat, feat_flat)

  psel = psel_flat.reshape(B, NPAD)[:, :N].reshape(B, 1, N)
  vals = pl.pallas_call(
      _mlp_kernel,
      grid=(B,),
      in_specs=[
          pl.BlockSpec((1, C, N), lambda b: (b, 0, 0)),
          pl.BlockSpec((C, 256), lambda b: (0, 0)),
          pl.BlockSpec((1, 256), lambda b: (0, 0)),
          pl.BlockSpec((1, 256), lambda b: (0, 0)),
          pl.BlockSpec((1, 1), lambda b: (0, 0)),
          pl.BlockSpec((1, 1, N), lambda b: (b, 0, 0)),
      ],
      out_specs=pl.BlockSpec((1, 1, N), lambda b: (b, 0, 0)),
      out_shape=jax.ShapeDtypeStruct((B, 1, N), jnp.float32),
  )(featsT, W1, b1.reshape(1, 256), W2.reshape(1, 256), b2.reshape(1, 1),
    psel)

  scatter = pl.kernel(
      _scatter_kernel,
      out_type=jax.ShapeDtypeStruct((B * HW,), jnp.float32),
      mesh=_sc_mesh,
      scratch_types=[
          pltpu.VMEM((CHUNK,), jnp.float32),
          pltpu.VMEM((N // NTILES,), jnp.int32),
          pltpu.VMEM((N // NTILES,), jnp.float32),
          pltpu.VMEM((N // NTILES,), jnp.int32),
      ],
      compiler_params=_sc_params,
  )
  yu = scatter(pf.reshape(B * HW), pix_flat, vals.reshape(B * N))
  return (A, yu.reshape(B, 1, H, W))


# speedup vs baseline: 18.5196x; 1.1588x over previous
import functools

import jax
import jax.numpy as jnp
from jax import lax
from jax.experimental import pallas as pl
from jax.experimental.pallas import tpu as pltpu
from jax.experimental.pallas import tpu_sc as plsc

B, C, H, W = 4, 96, 384, 384
HW = H * W
S = HW // 128
TOP_K, TOP_L, TOP_I = 512, 128, 128
N = TOP_K + TOP_L + TOP_I
NPAD = N + 16
MAXBITS = 0x3F800000
NTILES = 16
CHUNK = HW // NTILES
NV = CHUNK // 16
SLOT_BASE = (0, TOP_K, TOP_K + TOP_L)
SEL_COUNT = (TOP_K, TOP_L, TOP_I)


def _sel_kernel(p_ref, a_ref, thr_ref, pref_ref):
  p = p_ref[...]
  conf = jnp.abs(2.0 * p - 1.0)
  a_ref[...] = conf
  unc_bits = lax.bitcast_convert_type(1.0 - conf, jnp.int32)
  p_bits = lax.bitcast_convert_type(p, jnp.int32)
  idx = (
      jax.lax.broadcasted_iota(jnp.int32, (S, 128), 0) * 128
      + jax.lax.broadcasted_iota(jnp.int32, (S, 128), 1)
  )[None]

  def cnt(pred):
    return jnp.sum(pred.astype(jnp.int32), axis=(1, 2))

  z = jnp.zeros((B,), jnp.int32)
  mb = jnp.full((B,), MAXBITS, jnp.int32)

  def body(_, st):
    lk, hk, ll, hl, li, hi_ = st
    mk = lk + (hk - lk + 1) // 2
    ml = ll + (hl - ll + 1) // 2
    mi = li + (hi_ - li) // 2
    okk = cnt(unc_bits >= mk[:, None, None]) >= TOP_K
    okl = cnt(p_bits >= ml[:, None, None]) >= TOP_L
    oki = cnt(p_bits <= mi[:, None, None]) >= TOP_I
    return (jnp.where(okk, mk, lk), jnp.where(okk, hk, mk - 1),
            jnp.where(okl, ml, ll), jnp.where(okl, hl, ml - 1),
            jnp.where(oki, li, mi + 1), jnp.where(oki, mi, hi_))

  tk, _, tl, _, _, ti = lax.fori_loop(0, 30, body, (z, mb, z, mb, z, mb))

  gk = cnt(unc_bits > tk[:, None, None])
  gl = cnt(p_bits > tl[:, None, None])
  gi = cnt(p_bits < ti[:, None, None])
  eqk = unc_bits == tk[:, None, None]
  eql = p_bits == tl[:, None, None]
  eqi = p_bits == ti[:, None, None]
  mh = jnp.full((B,), HW - 1, jnp.int32)

  def body2(_, st):
    lk, hk, ll, hl, li, hi_ = st
    mk = lk + (hk - lk) // 2
    ml = ll + (hl - ll) // 2
    mi = li + (hi_ - li) // 2
    okk = gk + cnt(eqk & (idx <= mk[:, None, None])) >= TOP_K
    okl = gl + cnt(eql & (idx <= ml[:, None, None])) >= TOP_L
    oki = gi + cnt(eqi & (idx <= mi[:, None, None])) >= TOP_I
    return (jnp.where(okk, lk, mk + 1), jnp.where(okk, mk, hk),
            jnp.where(okl, ll, ml + 1), jnp.where(okl, ml, hl),
            jnp.where(oki, li, mi + 1), jnp.where(oki, mi, hi_))

  _, ik, _, il, _, ii = lax.fori_loop(0, 18, body2,
                                      (z, mh, z, mh, z, mh))

  lane = jax.lax.broadcasted_iota(jnp.int32, (B, 128), 1)
  out = jnp.zeros((B, 128), jnp.int32)
  for j, v in enumerate([tk, ik, tl, il, ti, ii]):
    out = jnp.where(lane == j, v[:, None], out)
  thr_ref[...] = out

  def chunk_prefix(bits, t, i, largest):
    if largest:
      sel = (bits > t[:, None, None]) | ((bits == t[:, None, None])
                                         & (idx <= i[:, None, None]))
    else:
      sel = (bits < t[:, None, None]) | ((bits == t[:, None, None])
                                         & (idx <= i[:, None, None]))
    seli = sel.astype(jnp.int32)
    rows_per_chunk = CHUNK // 128
    out = jnp.zeros((B, 128), jnp.int32)
    run = jnp.zeros((B,), jnp.int32)
    for t in range(NTILES):
      out = jnp.where(lane == t, run[:, None], out)
      run = run + jnp.sum(
          seli[:, t * rows_per_chunk:(t + 1) * rows_per_chunk, :], axis=(1, 2))
    return out

  pref_ref[...] = jnp.concatenate([
      chunk_prefix(unc_bits, tk, ik, True),
      chunk_prefix(p_bits, tl, il, True),
      chunk_prefix(p_bits, ti, ii, False),
  ], axis=1)


def _compact_kernel(p_hbm, thr_hbm, pref_hbm, pix_hbm, psel_hbm, pvm, thr_s,
                    pref_s, ibuf0, ibuf1, ibuf2, pbuf0, pbuf1, pbuf2, offs_s):
  core = lax.axis_index("c")
  tile = lax.axis_index("s")
  lanes = lax.iota(jnp.int32, 16)
  ibufs = (ibuf0, ibuf1, ibuf2)
  pbufs = (pbuf0, pbuf1, pbuf2)

  for b_local in range(2):
    b = 2 * core + b_local
    pltpu.sync_copy(p_hbm.at[b, pl.ds(tile * CHUNK, CHUNK)], pvm)
    pltpu.sync_copy(thr_hbm.at[b], thr_s)
    thr_vec = thr_s[pl.ds(0, 16)]
    tk, ik = thr_vec[0], thr_vec[1]
    tl, il = thr_vec[2], thr_vec[3]
    ti, ii = thr_vec[4], thr_vec[5]
    tile_base = tile * CHUNK

    offs_s[0] = 0
    offs_s[1] = 0
    offs_s[2] = 0

    @pl.loop(0, NV)
    def _(i):
      v = pvm[pl.ds(i * 16, 16)]
      pb = plsc.bitcast(v, jnp.int32)
      ub = plsc.bitcast(1.0 - jnp.abs(2.0 * v - 1.0), jnp.int32)
      gi = tile_base + i * 16 + lanes
      m0 = (ub > tk) | ((ub == tk) & (gi <= ik))
      m1 = (pb > tl) | ((pb == tl) & (gi <= il))
      m2 = (pb < ti) | ((pb == ti) & (gi <= ii))
      nz = plsc.all_reduce_population_count(m0 | m1 | m2)

      @pl.when(nz[0] > 0)
      def _():
        o0, o1, o2 = offs_s[0], offs_s[1], offs_s[2]
        plsc.store_compressed(ibuf0.at[pl.ds(o0, 16)], gi, mask=m0)
        plsc.store_compressed(pbuf0.at[pl.ds(o0, 16)], v, mask=m0)
        plsc.store_compressed(ibuf1.at[pl.ds(o1, 16)], gi, mask=m1)
        plsc.store_compressed(pbuf1.at[pl.ds(o1, 16)], v, mask=m1)
        plsc.store_compressed(ibuf2.at[pl.ds(o2, 16)], gi, mask=m2)
        plsc.store_compressed(pbuf2.at[pl.ds(o2, 16)], v, mask=m2)
        offs_s[0] = o0 + plsc.all_reduce_population_count(m0)[0]
        offs_s[1] = o1 + plsc.all_reduce_population_count(m1)[0]
        offs_s[2] = o2 + plsc.all_reduce_population_count(m2)[0]

    counts = (offs_s[0], offs_s[1], offs_s[2])

    for s in range(3):
      pltpu.sync_copy(pref_hbm.at[pl.ds((b * 3 + s) * 128, 128)], pref_s)
      pre_vec = pref_s[pl.ds(0, 16)]
      prefix = jnp.sum(jnp.where(lanes == tile, pre_vec, 0))
      start = b * NPAD + SLOT_BASE[s] + prefix
      dump = b * NPAD + N + lanes
      cnt_s = counts[s]

      def wr_body(j, _, s=s, start=start, dump=dump, cnt_s=cnt_s):
        lanepos = j * 16 + lanes
        tgt = jnp.where(lanepos < cnt_s, start + lanepos, dump)
        pltpu.sync_copy(ibufs[s].at[pl.ds(j * 16, 16)], pix_hbm.at[tgt])
        pltpu.sync_copy(pbufs[s].at[pl.ds(j * 16, 16)], psel_hbm.at[tgt])
        return 0

      lax.fori_loop(0, (cnt_s + 15) // 16, wr_body, 0)


def _gather_kernel(pix_hbm, feat_hbm, ft_hbm, idxbuf, fvm, sem):
  core = lax.axis_index("c")
  tile = lax.axis_index("s")
  for b_local in range(2):
    b = 2 * core + b_local
    for j in range(N // 128):
      pltpu.sync_copy(pix_hbm.at[pl.ds(b * NPAD + 128 * j, 128)],
                      idxbuf.at[j])
    copies = []
    for c_local in range(C // NTILES):
      ch = c_local * NTILES + tile
      base_off = (b * C + ch) * HW
      for j in range(N // 128):
        copies.append(pltpu.async_copy(
            feat_hbm.at[pl.ds(base_off, HW)].at[idxbuf.at[j]],
            fvm.at[c_local, pl.ds(j * 128, 128)], sem))
    for cp in copies:
      cp.wait()
    writes = []
    for c_local in range(C // NTILES):
      ch = c_local * NTILES + tile
      writes.append(pltpu.async_copy(fvm.at[c_local], ft_hbm.at[b, ch], sem))
    for cp in writes:
      cp.wait()


def _mlp_kernel(ft_ref, w1_ref, b1_ref, w2_ref, b2_ref, psel_ref, vals_ref):
  ft = ft_ref[0]
  h = lax.dot_general(ft, w1_ref[...], (((0,), (0,)), ((), ())),
                      precision=lax.Precision.HIGHEST,
                      preferred_element_type=jnp.float32)
  h = jnp.maximum(h + b1_ref[...], 0.0)
  lg = lax.dot_general(w2_ref[...], h, (((1,), (1,)), ((), ())),
                       precision=lax.Precision.HIGHEST,
                       preferred_element_type=jnp.float32)
  prob = jax.nn.sigmoid(lg + b2_ref[...])
  psel = psel_ref[0]
  vals_ref[0] = jnp.abs(2.0 * psel - 1.0) * psel + prob


def _scatter_kernel(p_hbm, pix_hbm, vals_hbm, yu_hbm, pvm, pixvm, valsvm,
                    idxbuf):
  core = lax.axis_index("c")
  tile = lax.axis_index("s")
  lanes = lax.iota(jnp.int32, 16)

  for b_local in range(2):
    b = 2 * core + b_local
    pltpu.sync_copy(p_hbm.at[pl.ds(b * HW + tile * CHUNK, CHUNK)], pvm)

    @pl.loop(0, NV)
    def _(i):
      v = pvm[pl.ds(i * 16, 16)]
      pvm[pl.ds(i * 16, 16)] = jnp.abs(2.0 * v - 1.0) * v

    pltpu.sync_copy(pvm, yu_hbm.at[pl.ds(b * HW + tile * CHUNK, CHUNK)])

  plsc.subcore_barrier()

  NS = N // NTILES
  for b_local in range(2):
    b = 2 * core + b_local
    pltpu.sync_copy(pix_hbm.at[pl.ds(b * NPAD + tile * NS, NS)], pixvm)
    pltpu.sync_copy(vals_hbm.at[pl.ds(b * N + tile * NS, NS)], valsvm)
    for l in range(NS // 16):
      tgt = pixvm[pl.ds(l * 16, 16)] + b * HW
      pltpu.sync_copy(valsvm.at[pl.ds(l * 16, 16)], yu_hbm.at[tgt])


_sc_mesh = plsc.VectorSubcoreMesh(core_axis_name="c", subcore_axis_name="s",
                                  num_cores=2, num_subcores=16)
_sc_params = pltpu.CompilerParams(needs_layout_passes=False)


@jax.jit
def kernel(p_coarse_mask, feature_map, W1, b1, W2, b2):
  p2 = p_coarse_mask.reshape(B, S, 128)
  a3, thr, pref = pl.pallas_call(
      _sel_kernel,
      out_shape=(
          jax.ShapeDtypeStruct((B, S, 128), jnp.float32),
          jax.ShapeDtypeStruct((B, 128), jnp.int32),
          jax.ShapeDtypeStruct((B, 384), jnp.int32),
      ),
  )(p2)
  A = a3.reshape(B, 1, H, W)

  pf = p_coarse_mask.reshape(B, HW)
  feat_flat = feature_map.reshape(B * C * HW)

  compact = pl.kernel(
      _compact_kernel,
      out_type=(
          jax.ShapeDtypeStruct((B * NPAD,), jnp.int32),
          jax.ShapeDtypeStruct((B * NPAD,), jnp.float32),
      ),
      mesh=_sc_mesh,
      scratch_types=[
          pltpu.VMEM((CHUNK,), jnp.float32),
          pltpu.VMEM((128,), jnp.int32),
          pltpu.VMEM((128,), jnp.int32),
          pltpu.VMEM((TOP_K + 16,), jnp.int32),
          pltpu.VMEM((TOP_L + 16,), jnp.int32),
          pltpu.VMEM((TOP_I + 16,), jnp.int32),
          pltpu.VMEM((TOP_K + 16,), jnp.float32),
          pltpu.VMEM((TOP_L + 16,), jnp.float32),
          pltpu.VMEM((TOP_I + 16,), jnp.float32),
          pltpu.SMEM((8,), jnp.int32),
      ],
      compiler_params=_sc_params,
  )
  pix_flat, psel_flat = compact(pf, thr, pref.reshape(B * 384))

  gather = pl.kernel(
      _gather_kernel,
      out_type=jax.ShapeDtypeStruct((B, C, N), jnp.float32),
      mesh=_sc_mesh,
      scratch_types=[
          pltpu.VMEM((N // 128, 128), jnp.int32),
          pltpu.VMEM((C // NTILES, N), jnp.float32),
          pltpu.SemaphoreType.DMA,
      ],
      compiler_params=_sc_params,
  )
  featsT = gather(pix_flat, feat_flat)

  psel = psel_flat.reshape(B, NPAD)[:, :N].reshape(B, 1, N)
  vals = pl.pallas_call(
      _mlp_kernel,
      grid=(B,),
      in_specs=[
          pl.BlockSpec((1, C, N), lambda b: (b, 0, 0)),
          pl.BlockSpec((C, 256), lambda b: (0, 0)),
          pl.BlockSpec((1, 256), lambda b: (0, 0)),
          pl.BlockSpec((1, 256), lambda b: (0, 0)),
          pl.BlockSpec((1, 1), lambda b: (0, 0)),
          pl.BlockSpec((1, 1, N), lambda b: (b, 0, 0)),
      ],
      out_specs=pl.BlockSpec((1, 1, N), lambda b: (b, 0, 0)),
      out_shape=jax.ShapeDtypeStruct((B, 1, N), jnp.float32),
  )(featsT, W1, b1.reshape(1, 256), W2.reshape(1, 256), b2.reshape(1, 1),
    psel)

  scatter = pl.kernel(
      _scatter_kernel,
      out_type=jax.ShapeDtypeStruct((B * HW,), jnp.float32),
      mesh=_sc_mesh,
      scratch_types=[
          pltpu.VMEM((CHUNK,), jnp.float32),
          pltpu.VMEM((N // NTILES,), jnp.int32),
          pltpu.VMEM((N // NTILES,), jnp.float32),
          pltpu.VMEM((N // NTILES,), jnp.int32),
      ],
      compiler_params=_sc_params,
  )
  yu = scatter(pf.reshape(B * HW), pix_flat, vals.reshape(B * N))
  return (A, yu.reshape(B, 1, H, W))
